# Initial kernel scaffold; baseline (speedup 1.0000x reference)
import jax
import jax.numpy as jnp
from jax.experimental import pallas as pl


def kernel(x, edge_index, W1, b1, W_mu, b_mu, W_ls, b_ls, W_n, b_n, W_n2, b_n2):
    raise NotImplementedError("write your pallas kernel here")


# speedup vs baseline: 16.6440x; 16.6440x over previous
import jax
import jax.numpy as jnp
from jax import lax
from jax.experimental import pallas as pl
from jax.experimental.pallas import tpu as pltpu
from jax.experimental.pallas import tpu_sc as plsc

N = 10000
E = 320000
N_PAD = 10240
CH = 100
BLK = 40
ROWS_PER_TILE = (E // CH) // 16
NBLK = ROWS_PER_TILE // BLK
RPS = N_PAD // 16
ZR = 128


def _make_prop(W, gather):
    mesh = plsc.VectorSubcoreMesh(core_axis_name="c", subcore_axis_name="s")
    qn = W // 16

    scratch_types = ([pltpu.VMEM((BLK, CH), jnp.int32)] if gather else []) + [
        pltpu.VMEM((BLK, CH), jnp.int32),
        pltpu.VMEM((CH, W), jnp.float32),
        pltpu.VMEM((ZR, W), jnp.float32),
        pltpu.VMEM_SHARED((N_PAD, W), jnp.float32),
        pltpu.SemaphoreType.DMA,
    ]

    def body(*refs):
        if gather:
            table, srcx, dst2d, out, sbuf, dbuf, rows, zbuf, acc, sem = refs
        else:
            dst2d, out, dbuf, rows, zbuf, acc, sem = refs
        c = lax.axis_index("c")
        s = lax.axis_index("s")

        def zstore(i, _):
            zbuf[i // qn, pl.ds((i % qn) * 16, 16)] = jnp.zeros((16,), jnp.float32)
            return 0
        lax.fori_loop(0, ZR * qn, zstore, 0)
        if not gather:
            def ostore(i, _):
                rows[i // qn, pl.ds((i % qn) * 16, 16)] = jnp.ones((16,), jnp.float32)
                return 0
            lax.fori_loop(0, CH * qn, ostore, 0)
        for t in range(RPS // ZR):
            pltpu.sync_copy(zbuf, acc.at[pl.ds(s * RPS + t * ZR, ZR)])
        plsc.subcore_barrier()

        drow0 = s * ROWS_PER_TILE
        srow0 = c * (E // CH) + s * ROWS_PER_TILE

        def blk_body(b, _):
            if gather:
                pltpu.sync_copy(srcx.at[pl.ds(srow0 + b * BLK, BLK)], sbuf)
            pltpu.sync_copy(dst2d.at[pl.ds(drow0 + b * BLK, BLK)], dbuf)

            def chunk(j, _):
                if gather:
                    pltpu.async_copy(table.at[sbuf.at[j]], rows, sem).wait()
                pltpu.sync_copy(rows, acc.at[dbuf.at[j]], add=True)
                return 0
            lax.fori_loop(0, BLK, chunk, 0)
            return 0
        lax.fori_loop(0, NBLK, blk_body, 0)

        plsc.subcore_barrier()
        for t in range(RPS // ZR):
            pltpu.sync_copy(acc.at[pl.ds(s * RPS + t * ZR, ZR)],
                            out.at[c, pl.ds(s * RPS + t * ZR, ZR)])

    return pl.kernel(
        body,
        out_type=jax.ShapeDtypeStruct((2, N_PAD, W), jnp.float32),
        mesh=mesh,
        scratch_types=scratch_types,
        compiler_params=pltpu.CompilerParams(use_tc_tiling_on_sc=False),
    )


_deg_kernel = _make_prop(16, gather=False)
_prop64 = _make_prop(64, gather=True)
_prop128 = _make_prop(128, gather=True)
_prop16 = _make_prop(16, gather=True)


def _tc1(degp, x):
    def body(degp_ref, x_ref, v0_ref, dinv_ref):
        deg = degp_ref[0, 0:N, 0:1] + 1.0
        dinv = lax.rsqrt(deg)
        xd = x_ref[...] * dinv
        v0_ref[...] = jnp.concatenate([xd[:, :64], xd[:, 64:]], axis=0)
        dinv_ref[...] = dinv
    return pl.pallas_call(body, out_shape=[
        jax.ShapeDtypeStruct((2 * N, 64), jnp.float32),
        jax.ShapeDtypeStruct((N, 1), jnp.float32),
    ])(degp, x)


_RB = 2000
_NB = N // _RB


def _tc2(acc0, v0s, dinv, W1, b1, Wn, bn):
    def body(a_ref, va_ref, vb_ref, d_ref, w1_ref, b1_ref, wn_ref, bn_ref,
             out_ref, h_ref):
        dinv = d_ref[...]
        s0 = (jnp.concatenate([a_ref[0], a_ref[1]], axis=1)
              + jnp.concatenate([va_ref[...], vb_ref[...]], axis=1))
        agg0 = dinv * s0
        xn = jnp.maximum(
            jnp.dot(agg0, w1_ref[...], preferred_element_type=jnp.float32)
            + b1_ref[...], 0.0)
        h = jnp.maximum(
            jnp.dot(agg0, wn_ref[...], preferred_element_type=jnp.float32)
            + bn_ref[...], 0.0)
        out_ref[...] = dinv * xn
        h_ref[...] = dinv * jnp.concatenate(
            [h, jnp.zeros((_RB, 13), jnp.float32)], axis=1)
    return pl.pallas_call(
        body,
        grid=(2, _NB),
        in_specs=[
            pl.BlockSpec((2, _RB, 64), lambda j, i: (0, i, 0)),
            pl.BlockSpec((_RB, 64), lambda j, i: (i, 0)),
            pl.BlockSpec((_RB, 64), lambda j, i: (_NB + i, 0)),
            pl.BlockSpec((_RB, 1), lambda j, i: (i, 0)),
            pl.BlockSpec((128, 128), lambda j, i: (0, j)),
            pl.BlockSpec((1, 128), lambda j, i: (0, j)),
            pl.BlockSpec((128, 3), lambda j, i: (0, 0)),
            pl.BlockSpec((1, 3), lambda j, i: (0, 0)),
        ],
        out_specs=[
            pl.BlockSpec((_RB, 128), lambda j, i: (j * _NB + i, 0)),
            pl.BlockSpec((_RB, 16), lambda j, i: (j * _NB + i, 0)),
        ],
        out_shape=[
            jax.ShapeDtypeStruct((2 * N, 128), jnp.float32),
            jax.ShapeDtypeStruct((2 * N, 16), jnp.float32),
        ])(acc0[:, 0:N], v0s, v0s, dinv, W1, b1, Wn, bn)


def _tc3(acc1, acc2, v1s, hs, dinv, Wmu, bmu, Wls, bls, Wn2, bn2):
    def body(a1_ref, a2_ref, va_ref, vb_ref, h_ref, d_ref, wmu_ref, bmu_ref,
             wls_ref, bls_ref, wn2_ref, bn2_ref, mu_ref, ls_ref, node_ref):
        dinv = d_ref[...]
        ga = dinv * (a1_ref[0] + va_ref[...])
        gb = dinv * (a1_ref[1] + vb_ref[...])
        g2 = (dinv * (a2_ref[0] + h_ref[...]))[:, 0:3]
        mu_ref[...] = (
            jnp.dot(ga, wmu_ref[:128], preferred_element_type=jnp.float32)
            + jnp.dot(gb, wmu_ref[128:], preferred_element_type=jnp.float32)
            + bmu_ref[...])
        ls_ref[...] = (
            jnp.dot(ga, wls_ref[:128], preferred_element_type=jnp.float32)
            + jnp.dot(gb, wls_ref[128:], preferred_element_type=jnp.float32)
            + bls_ref[...])
        node_ref[...] = (
            jnp.dot(g2, wn2_ref[...], preferred_element_type=jnp.float32)
            + bn2_ref[...])
    return pl.pallas_call(
        body,
        grid=(_NB,),
        in_specs=[
            pl.BlockSpec((2, _RB, 128), lambda i: (0, i, 0)),
            pl.BlockSpec((2, _RB, 16), lambda i: (0, i, 0)),
            pl.BlockSpec((_RB, 128), lambda i: (i, 0)),
            pl.BlockSpec((_RB, 128), lambda i: (_NB + i, 0)),
            pl.BlockSpec((_RB, 16), lambda i: (i, 0)),
            pl.BlockSpec((_RB, 1), lambda i: (i, 0)),
            pl.BlockSpec((256, 128), lambda i: (0, 0)),
            pl.BlockSpec((1, 128), lambda i: (0, 0)),
            pl.BlockSpec((256, 128), lambda i: (0, 0)),
            pl.BlockSpec((1, 128), lambda i: (0, 0)),
            pl.BlockSpec((3, 6), lambda i: (0, 0)),
            pl.BlockSpec((1, 6), lambda i: (0, 0)),
        ],
        out_specs=[
            pl.BlockSpec((_RB, 128), lambda i: (i, 0)),
            pl.BlockSpec((_RB, 128), lambda i: (i, 0)),
            pl.BlockSpec((_RB, 6), lambda i: (i, 0)),
        ],
        out_shape=[
        jax.ShapeDtypeStruct((N, 128), jnp.float32),
        jax.ShapeDtypeStruct((N, 128), jnp.float32),
        jax.ShapeDtypeStruct((N, 6), jnp.float32),
    ])(acc1[:, 0:N], acc2[:, 0:N], v1s, v1s, hs, dinv,
       Wmu, bmu, Wls, bls, Wn2, bn2)


def kernel(x, edge_index, W1, b1, W_mu, b_mu, W_ls, b_ls, W_n, b_n, W_n2, b_n2):
    src = edge_index[0].astype(jnp.int32)
    dst = edge_index[1].astype(jnp.int32)
    srcx2d = jnp.concatenate([src, src + N]).reshape(2 * E // CH, CH)
    dst2d = dst.reshape(E // CH, CH)

    degp = _deg_kernel(dst2d)
    v0s, dinv = _tc1(degp, x)
    acc0 = _prop64(v0s, srcx2d, dst2d)
    v1s, hs = _tc2(acc0, v0s, dinv, W1, b1.reshape(1, -1),
                   W_n, b_n.reshape(1, -1))
    acc1 = _prop128(v1s, srcx2d, dst2d)
    acc2 = _prop16(hs, srcx2d, dst2d)
    return _tc3(acc1, acc2, v1s, hs, dinv, W_mu, b_mu.reshape(1, -1),
                W_ls, b_ls.reshape(1, -1), W_n2, b_n2.reshape(1, -1))


# speedup vs baseline: 16.9055x; 1.0157x over previous
import jax
import jax.numpy as jnp
from jax import lax
from jax.experimental import pallas as pl
from jax.experimental.pallas import tpu as pltpu
from jax.experimental.pallas import tpu_sc as plsc

N = 10000
E = 320000
N_PAD = 10240
CH = 100
ROWS_PER_TILE = (E // CH) // 16
RPS = N_PAD // 16
ZR = 128


def _make_prop(W, gather, trows):
    mesh = plsc.VectorSubcoreMesh(core_axis_name="c", subcore_axis_name="s")
    qn = W // 16
    RPT = ROWS_PER_TILE

    scratch_types = ([pltpu.VMEM((RPT, CH), jnp.int32)] if gather else []) + [
        pltpu.VMEM((RPT, CH), jnp.int32),
        pltpu.VMEM((CH, W), jnp.float32),
        pltpu.VMEM((CH, W), jnp.float32),
        pltpu.VMEM((ZR, W), jnp.float32),
        pltpu.VMEM_SHARED((N_PAD, W), jnp.float32),
        pltpu.SemaphoreType.DMA,
        pltpu.SemaphoreType.DMA,
        pltpu.SemaphoreType.DMA,
    ]

    def body(*refs):
        if gather:
            (table, srcx, dst2d, out,
             sidx, didx, rows0, rows1, zbuf, acc, sem0, sem1, semz) = refs
        else:
            (dst2d, out,
             didx, rows0, rows1, zbuf, acc, sem0, sem1, semz) = refs
        c = lax.axis_index("c")
        s = lax.axis_index("s")

        def zstore(i, _):
            zbuf[i // qn, pl.ds((i % qn) * 16, 16)] = jnp.zeros((16,), jnp.float32)
            return 0
        lax.fori_loop(0, ZR * qn, zstore, 0)
        for t in range(RPS // ZR):
            pltpu.async_copy(zbuf, acc.at[pl.ds(s * RPS + t * ZR, ZR)], semz)
        if not gather:
            for r in (rows0, rows1):
                def ostore(i, _, _r=r):
                    _r[i // qn, pl.ds((i % qn) * 16, 16)] = jnp.ones((16,), jnp.float32)
                    return 0
                lax.fori_loop(0, CH * qn, ostore, 0)

        drow0 = s * RPT
        pltpu.sync_copy(dst2d.at[pl.ds(drow0, RPT)], didx)
        if gather:
            srow0 = c * (E // CH) + s * RPT
            pltpu.sync_copy(srcx.at[pl.ds(srow0, RPT)], sidx)
        for t in range(RPS // ZR):
            pltpu.make_async_copy(zbuf, acc.at[pl.ds(s * RPS + t * ZR, ZR)],
                                  semz).wait()
        plsc.subcore_barrier()

        if gather:
            pltpu.async_copy(table.at[sidx.at[0]], rows0, sem0)

            def pair(b, _):
                j0 = 2 * b
                pltpu.make_async_copy(table.at[sidx.at[j0]], rows0, sem0).wait()
                pltpu.async_copy(table.at[sidx.at[j0 + 1]], rows1, sem1)
                pltpu.sync_copy(rows0, acc.at[didx.at[j0]], add=True)
                pltpu.make_async_copy(table.at[sidx.at[j0 + 1]], rows1,
                                      sem1).wait()
                nxt = jnp.minimum(j0 + 2, RPT - 1)
                pltpu.async_copy(table.at[sidx.at[nxt]], rows0, sem0)
                pltpu.sync_copy(rows1, acc.at[didx.at[j0 + 1]], add=True)
                return 0
            lax.fori_loop(0, RPT // 2, pair, 0)
            pltpu.make_async_copy(table.at[sidx.at[RPT - 1]], rows0,
                                  sem0).wait()
        else:
            def pair(b, _):
                pltpu.sync_copy(rows0, acc.at[didx.at[2 * b]], add=True)
                pltpu.sync_copy(rows1, acc.at[didx.at[2 * b + 1]], add=True)
                return 0
            lax.fori_loop(0, RPT // 2, pair, 0)

        plsc.subcore_barrier()
        for t in range(RPS // ZR):
            pltpu.async_copy(acc.at[pl.ds(s * RPS + t * ZR, ZR)],
                             out.at[c, pl.ds(s * RPS + t * ZR, ZR)], semz)
        for t in range(RPS // ZR):
            pltpu.make_async_copy(acc.at[pl.ds(s * RPS + t * ZR, ZR)],
                                  out.at[c, pl.ds(s * RPS + t * ZR, ZR)],
                                  semz).wait()

    return pl.kernel(
        body,
        out_type=jax.ShapeDtypeStruct((2, N_PAD, W), jnp.float32),
        mesh=mesh,
        scratch_types=scratch_types,
        compiler_params=pltpu.CompilerParams(use_tc_tiling_on_sc=False),
    )


_deg_kernel = _make_prop(16, gather=False, trows=None)
_gp16 = _make_prop(16, gather=True, trows=2 * N)
_gp64 = _make_prop(64, gather=True, trows=4 * N)

_RB = 2000
_NB = N // _RB


def _tc1(degcol, x):
    def body(deg_ref, x_ref, v0_ref, dinv_ref):
        dinv = lax.rsqrt(deg_ref[...] + 1.0)
        xd = x_ref[...] * dinv
        v0_ref[...] = jnp.concatenate(
            [xd[:, :64], xd[:, 64:], jnp.zeros((2 * N, 64), jnp.float32)],
            axis=0)
        dinv_ref[...] = dinv
    return pl.pallas_call(body, out_shape=[
        jax.ShapeDtypeStruct((4 * N, 64), jnp.float32),
        jax.ShapeDtypeStruct((N, 1), jnp.float32),
    ])(degcol, x)


def _tc2(acc0, v0s, dinv, W1, b1, Wn, bn):
    def body(a_ref, va_ref, vb_ref, d_ref, w1_ref, b1_ref, wn_ref, bn_ref,
             out_ref, h_ref):
        dinv = d_ref[...]
        s0 = (jnp.concatenate([a_ref[0], a_ref[1]], axis=1)
              + jnp.concatenate([va_ref[...], vb_ref[...]], axis=1))
        agg0 = dinv * s0
        xn = jnp.maximum(
            jnp.dot(agg0, w1_ref[0], preferred_element_type=jnp.float32)
            + b1_ref[0], 0.0)
        h = jnp.maximum(
            jnp.dot(agg0, wn_ref[...], preferred_element_type=jnp.float32)
            + bn_ref[...], 0.0)
        out_ref[...] = dinv * xn
        h_ref[...] = dinv * jnp.concatenate(
            [h, jnp.zeros((_RB, 13), jnp.float32)], axis=1)
    return pl.pallas_call(
        body,
        grid=(4, _NB),
        in_specs=[
            pl.BlockSpec((2, _RB, 64), lambda j, i: (0, i, 0)),
            pl.BlockSpec((_RB, 64), lambda j, i: (i, 0)),
            pl.BlockSpec((_RB, 64), lambda j, i: (_NB + i, 0)),
            pl.BlockSpec((_RB, 1), lambda j, i: (i, 0)),
            pl.BlockSpec((1, 128, 64), lambda j, i: (j, 0, 0)),
            pl.BlockSpec((1, 1, 64), lambda j, i: (j, 0, 0)),
            pl.BlockSpec((128, 3), lambda j, i: (0, 0)),
            pl.BlockSpec((1, 3), lambda j, i: (0, 0)),
        ],
        out_specs=[
            pl.BlockSpec((_RB, 64), lambda j, i: (j * _NB + i, 0)),
            pl.BlockSpec((_RB, 16), lambda j, i: ((j % 2) * _NB + i, 0)),
        ],
        out_shape=[
            jax.ShapeDtypeStruct((4 * N, 64), jnp.float32),
            jax.ShapeDtypeStruct((2 * N, 16), jnp.float32),
        ])(acc0[:, 0:N], v0s, v0s, dinv, W1, b1, Wn, bn)


def _tc3(acc1a, acc1b, acc2, xnq, hs, dinv, Wmu, bmu, Wls, bls, Wn2, bn2):
    def body(a1_ref, b1_ref, a2_ref, x0_ref, x1_ref, x2_ref, x3_ref, h_ref,
             d_ref, wmu_ref, bmu_ref, wls_ref, bls_ref, wn2_ref, bn2_ref,
             mu_ref, ls_ref, node_ref):
        dinv = d_ref[...]
        ga = dinv * (jnp.concatenate([a1_ref[0], a1_ref[1]], axis=1)
                     + jnp.concatenate([x0_ref[...], x1_ref[...]], axis=1))
        gb = dinv * (jnp.concatenate([b1_ref[0], b1_ref[1]], axis=1)
                     + jnp.concatenate([x2_ref[...], x3_ref[...]], axis=1))
        g2 = (dinv * (a2_ref[0] + h_ref[...]))[:, 0:3]
        mu_ref[...] = (
            jnp.dot(ga, wmu_ref[:128], preferred_element_type=jnp.float32)
            + jnp.dot(gb, wmu_ref[128:], preferred_element_type=jnp.float32)
            + bmu_ref[...])
        ls_ref[...] = (
            jnp.dot(ga, wls_ref[:128], preferred_element_type=jnp.float32)
            + jnp.dot(gb, wls_ref[128:], preferred_element_type=jnp.float32)
            + bls_ref[...])
        node_ref[...] = (
            jnp.dot(g2, wn2_ref[...], preferred_element_type=jnp.float32)
            + bn2_ref[...])
    return pl.pallas_call(
        body,
        grid=(_NB,),
        in_specs=[
            pl.BlockSpec((2, _RB, 64), lambda i: (0, i, 0)),
            pl.BlockSpec((2, _RB, 64), lambda i: (0, i, 0)),
            pl.BlockSpec((2, _RB, 16), lambda i: (0, i, 0)),
            pl.BlockSpec((_RB, 64), lambda i: (i, 0)),
            pl.BlockSpec((_RB, 64), lambda i: (_NB + i, 0)),
            pl.BlockSpec((_RB, 64), lambda i: (2 * _NB + i, 0)),
            pl.BlockSpec((_RB, 64), lambda i: (3 * _NB + i, 0)),
            pl.BlockSpec((_RB, 16), lambda i: (i, 0)),
            pl.BlockSpec((_RB, 1), lambda i: (i, 0)),
            pl.BlockSpec((256, 128), lambda i: (0, 0)),
            pl.BlockSpec((1, 128), lambda i: (0, 0)),
            pl.BlockSpec((256, 128), lambda i: (0, 0)),
            pl.BlockSpec((1, 128), lambda i: (0, 0)),
            pl.BlockSpec((3, 6), lambda i: (0, 0)),
            pl.BlockSpec((1, 6), lambda i: (0, 0)),
        ],
        out_specs=[
            pl.BlockSpec((_RB, 128), lambda i: (i, 0)),
            pl.BlockSpec((_RB, 128), lambda i: (i, 0)),
            pl.BlockSpec((_RB, 6), lambda i: (i, 0)),
        ],
        out_shape=[
            jax.ShapeDtypeStruct((N, 128), jnp.float32),
            jax.ShapeDtypeStruct((N, 128), jnp.float32),
            jax.ShapeDtypeStruct((N, 6), jnp.float32),
        ])(acc1a[:, 0:N], acc1b[:, 0:N], acc2[:, 0:N], xnq, xnq, xnq, xnq,
           hs, dinv, Wmu, bmu, Wls, bls, Wn2, bn2)


def kernel(x, edge_index, W1, b1, W_mu, b_mu, W_ls, b_ls, W_n, b_n, W_n2, b_n2):
    src = edge_index[0].astype(jnp.int32)
    dst = edge_index[1].astype(jnp.int32)
    srcx2d = jnp.concatenate([src, src + N]).reshape(2 * E // CH, CH)
    dst2d = dst.reshape(E // CH, CH)

    degp = _deg_kernel(dst2d)
    v0s, dinv = _tc1(degp[0, 0:N, 0:1], x)
    acc0 = _gp64(v0s, srcx2d, dst2d)
    W1q = jnp.stack([W1[:, 64 * q:64 * (q + 1)] for q in range(4)])
    b1q = jnp.stack([b1[64 * q:64 * (q + 1)].reshape(1, -1) for q in range(4)])
    xnq, hs = _tc2(acc0, v0s, dinv, W1q, b1q, W_n, b_n.reshape(1, -1))
    acc1a = _gp64(xnq, srcx2d, dst2d)
    srcx2d_hi = srcx2d + 2 * N + (acc1a[0, 0, 0] * 0.0).astype(jnp.int32)
    acc1b = _gp64(xnq, srcx2d_hi, dst2d)
    acc2 = _gp16(hs, srcx2d, dst2d)
    return _tc3(acc1a, acc1b, acc2, xnq, hs, dinv, W_mu, b_mu.reshape(1, -1),
                W_ls, b_ls.reshape(1, -1), W_n2, b_n2.reshape(1, -1))


# speedup vs baseline: 25.5369x; 1.5106x over previous
import jax
import jax.numpy as jnp
from jax import lax
from jax.experimental import pallas as pl
from jax.experimental.pallas import tpu as pltpu
from jax.experimental.pallas import tpu_sc as plsc

N = 10000
E = 320000
N_PAD = 10240
CH = 100
ROWS_PER_TILE = (E // CH) // 16
RPS = N_PAD // 16
ZR = 128


def _make_prop(W, gather, trows):
    mesh = plsc.VectorSubcoreMesh(core_axis_name="c", subcore_axis_name="s")
    qn = W // 16
    RPT = ROWS_PER_TILE

    if gather:
        idx_scratch = [pltpu.VMEM((8, CH), jnp.int32) for _ in range(4)]
    else:
        idx_scratch = [pltpu.VMEM((RPT, CH), jnp.int32)]
    scratch_types = idx_scratch + [
        pltpu.VMEM((CH, W), jnp.float32) for _ in range(8)] + [
        pltpu.VMEM((ZR, W), jnp.float32),
        pltpu.VMEM_SHARED((N_PAD, W), jnp.float32),
        pltpu.SemaphoreType.DMA,
        pltpu.SemaphoreType.DMA,
        pltpu.SemaphoreType.DMA,
        pltpu.SemaphoreType.DMA,
        pltpu.SemaphoreType.DMA,
        pltpu.SemaphoreType.DMA,
    ]

    def body(*refs):
        if gather:
            (table, srcx, dst2d, out, sidxA, didxA, sidxB, didxB) = refs[:8]
            ring = refs[8:16]
            (zbuf, acc, gsa, gsb, ssa, ssb, semz, isem) = refs[16:]
        else:
            (dst2d, out, didx) = refs[:3]
            ring = refs[3:11]
            (zbuf, acc, gsa, gsb, ssa, ssb, semz, isem) = refs[11:]
        c = lax.axis_index("c")
        s = lax.axis_index("s")

        def zstore(i, _):
            zbuf[i // qn, pl.ds((i % qn) * 16, 16)] = jnp.zeros((16,), jnp.float32)
            return 0
        lax.fori_loop(0, ZR * qn, zstore, 0)
        for t in range(RPS // ZR):
            pltpu.async_copy(zbuf, acc.at[pl.ds(s * RPS + t * ZR, ZR)], semz)
        if not gather:
            for r in ring:
                def ostore(i, _, _r=r):
                    _r[i // qn, pl.ds((i % qn) * 16, 16)] = jnp.ones(
                        (16,), jnp.float32)
                    return 0
                lax.fori_loop(0, CH * qn, ostore, 0)

        drow0 = s * RPT
        if gather:
            srow0 = c * (E // CH) + s * RPT
            pltpu.sync_copy(srcx.at[pl.ds(srow0, 8)], sidxA)
            pltpu.sync_copy(dst2d.at[pl.ds(drow0, 8)], didxA)
            pltpu.async_copy(srcx.at[pl.ds(srow0 + 8, 8)], sidxB, isem)
            pltpu.async_copy(dst2d.at[pl.ds(drow0 + 8, 8)], didxB, isem)
        else:
            pltpu.sync_copy(dst2d.at[pl.ds(drow0, RPT)], didx)
        for t in range(RPS // ZR):
            pltpu.make_async_copy(zbuf, acc.at[pl.ds(s * RPS + t * ZR, ZR)],
                                  semz).wait()
        plsc.subcore_barrier()

        if gather:
            NBODY = RPT // 8
            for q in range(4):
                pltpu.async_copy(table.at[sidxA.at[q]], ring[q], gsa)

            def one_body(b, sidx, didx, sidx_o, didx_o):
                for q in range(4):
                    pltpu.make_async_copy(table.at[sidx.at[q]],
                                          ring[q], gsa).wait()
                for q in range(4):
                    pltpu.async_copy(ring[q], acc.at[didx.at[q]],
                                     ssa, add=True)
                @pl.when(b > 0)
                def _():
                    for q in range(4):
                        pltpu.make_async_copy(ring[4 + q],
                                              acc.at[didx_o.at[4 + q]],
                                              ssb).wait()
                    nrow = jnp.minimum(8 * (b + 1), RPT - 8)
                    pltpu.async_copy(srcx.at[pl.ds(srow0 + nrow, 8)],
                                     sidx_o, isem)
                    pltpu.async_copy(dst2d.at[pl.ds(drow0 + nrow, 8)],
                                     didx_o, isem)
                for q in range(4):
                    pltpu.async_copy(table.at[sidx.at[4 + q]],
                                     ring[4 + q], gsb)
                for q in range(4):
                    pltpu.make_async_copy(table.at[sidx.at[4 + q]],
                                          ring[4 + q], gsb).wait()
                for q in range(4):
                    pltpu.async_copy(ring[4 + q], acc.at[didx.at[4 + q]],
                                     ssb, add=True)
                for q in range(4):
                    pltpu.make_async_copy(ring[q],
                                          acc.at[didx.at[q]], ssa).wait()
                pltpu.make_async_copy(srcx.at[pl.ds(srow0, 8)], sidx_o,
                                      isem).wait()
                pltpu.make_async_copy(dst2d.at[pl.ds(drow0, 8)], didx_o,
                                      isem).wait()
                for q in range(4):
                    pltpu.async_copy(table.at[sidx_o.at[q]], ring[q], gsa)

            def bodyfn(b, _):
                @pl.when(b % 2 == 0)
                def _():
                    one_body(b, sidxA, didxA, sidxB, didxB)

                @pl.when(b % 2 == 1)
                def _():
                    one_body(b, sidxB, didxB, sidxA, didxA)
                return 0
            lax.fori_loop(0, NBODY, bodyfn, 0)
            for q in range(4):
                pltpu.make_async_copy(table.at[sidxA.at[q]],
                                      ring[q], gsa).wait()
            for q in range(4):
                pltpu.make_async_copy(ring[4 + q],
                                      acc.at[didxA.at[4 + q]],
                                      ssb).wait()
        else:
            def blk(b, _):
                for q in range(8):
                    pltpu.async_copy(ring[q], acc.at[didx.at[8 * b + q]],
                                     ssa, add=True)
                for q in range(8):
                    pltpu.make_async_copy(ring[q],
                                          acc.at[didx.at[8 * b + q]],
                                          ssa).wait()
                return 0
            lax.fori_loop(0, RPT // 8, blk, 0)

        plsc.subcore_barrier()
        for t in range(RPS // ZR):
            pltpu.async_copy(acc.at[pl.ds(s * RPS + t * ZR, ZR)],
                             out.at[c, pl.ds(s * RPS + t * ZR, ZR)], semz)
        for t in range(RPS // ZR):
            pltpu.make_async_copy(acc.at[pl.ds(s * RPS + t * ZR, ZR)],
                                  out.at[c, pl.ds(s * RPS + t * ZR, ZR)],
                                  semz).wait()

    return pl.kernel(
        body,
        out_type=jax.ShapeDtypeStruct((2, N_PAD, W), jnp.float32),
        mesh=mesh,
        scratch_types=scratch_types,
        compiler_params=pltpu.CompilerParams(use_tc_tiling_on_sc=False),
    )


_deg_kernel = _make_prop(16, gather=False, trows=None)
_gp16 = _make_prop(16, gather=True, trows=2 * N)
_gp64 = _make_prop(64, gather=True, trows=4 * N)

_RB = 2000
_NB = N // _RB


def _tc1(degcol, x):
    def body(deg_ref, x_ref, v0_ref, dinv_ref):
        dinv = lax.rsqrt(deg_ref[...] + 1.0)
        xd = x_ref[...] * dinv
        v0_ref[...] = jnp.concatenate(
            [xd[:, :64], xd[:, 64:], jnp.zeros((2 * N, 64), jnp.float32)],
            axis=0)
        dinv_ref[...] = dinv
    return pl.pallas_call(body, out_shape=[
        jax.ShapeDtypeStruct((4 * N, 64), jnp.float32),
        jax.ShapeDtypeStruct((N, 1), jnp.float32),
    ])(degcol, x)


def _tc2(acc0, v0s, dinv, W1, b1, Wn, bn):
    def body(a_ref, va_ref, vb_ref, d_ref, w1_ref, b1_ref, wn_ref, bn_ref,
             out_ref, h_ref):
        dinv = d_ref[...]
        s0 = (jnp.concatenate([a_ref[0], a_ref[1]], axis=1)
              + jnp.concatenate([va_ref[...], vb_ref[...]], axis=1))
        agg0 = dinv * s0
        xn = jnp.maximum(
            jnp.dot(agg0, w1_ref[0], preferred_element_type=jnp.float32)
            + b1_ref[0], 0.0)
        h = jnp.maximum(
            jnp.dot(agg0, wn_ref[...], preferred_element_type=jnp.float32)
            + bn_ref[...], 0.0)
        out_ref[...] = dinv * xn
        h_ref[...] = dinv * jnp.concatenate(
            [h, jnp.zeros((_RB, 13), jnp.float32)], axis=1)
    return pl.pallas_call(
        body,
        grid=(4, _NB),
        in_specs=[
            pl.BlockSpec((2, _RB, 64), lambda j, i: (0, i, 0)),
            pl.BlockSpec((_RB, 64), lambda j, i: (i, 0)),
            pl.BlockSpec((_RB, 64), lambda j, i: (_NB + i, 0)),
            pl.BlockSpec((_RB, 1), lambda j, i: (i, 0)),
            pl.BlockSpec((1, 128, 64), lambda j, i: (j, 0, 0)),
            pl.BlockSpec((1, 1, 64), lambda j, i: (j, 0, 0)),
            pl.BlockSpec((128, 3), lambda j, i: (0, 0)),
            pl.BlockSpec((1, 3), lambda j, i: (0, 0)),
        ],
        out_specs=[
            pl.BlockSpec((_RB, 64), lambda j, i: (j * _NB + i, 0)),
            pl.BlockSpec((_RB, 16), lambda j, i: ((j % 2) * _NB + i, 0)),
        ],
        out_shape=[
            jax.ShapeDtypeStruct((4 * N, 64), jnp.float32),
            jax.ShapeDtypeStruct((2 * N, 16), jnp.float32),
        ])(acc0[:, 0:N], v0s, v0s, dinv, W1, b1, Wn, bn)


def _tc3(acc1a, acc1b, acc2, xnq, hs, dinv, Wmu, bmu, Wls, bls, Wn2, bn2):
    def body(a1_ref, b1_ref, a2_ref, x0_ref, x1_ref, x2_ref, x3_ref, h_ref,
             d_ref, wmu_ref, bmu_ref, wls_ref, bls_ref, wn2_ref, bn2_ref,
             mu_ref, ls_ref, node_ref):
        dinv = d_ref[...]
        ga = dinv * (jnp.concatenate([a1_ref[0], a1_ref[1]], axis=1)
                     + jnp.concatenate([x0_ref[...], x1_ref[...]], axis=1))
        gb = dinv * (jnp.concatenate([b1_ref[0], b1_ref[1]], axis=1)
                     + jnp.concatenate([x2_ref[...], x3_ref[...]], axis=1))
        g2 = (dinv * (a2_ref[0] + h_ref[...]))[:, 0:3]
        mu_ref[...] = (
            jnp.dot(ga, wmu_ref[:128], preferred_element_type=jnp.float32)
            + jnp.dot(gb, wmu_ref[128:], preferred_element_type=jnp.float32)
            + bmu_ref[...])
        ls_ref[...] = (
            jnp.dot(ga, wls_ref[:128], preferred_element_type=jnp.float32)
            + jnp.dot(gb, wls_ref[128:], preferred_element_type=jnp.float32)
            + bls_ref[...])
        node_ref[...] = (
            jnp.dot(g2, wn2_ref[...], preferred_element_type=jnp.float32)
            + bn2_ref[...])
    return pl.pallas_call(
        body,
        grid=(_NB,),
        in_specs=[
            pl.BlockSpec((2, _RB, 64), lambda i: (0, i, 0)),
            pl.BlockSpec((2, _RB, 64), lambda i: (0, i, 0)),
            pl.BlockSpec((2, _RB, 16), lambda i: (0, i, 0)),
            pl.BlockSpec((_RB, 64), lambda i: (i, 0)),
            pl.BlockSpec((_RB, 64), lambda i: (_NB + i, 0)),
            pl.BlockSpec((_RB, 64), lambda i: (2 * _NB + i, 0)),
            pl.BlockSpec((_RB, 64), lambda i: (3 * _NB + i, 0)),
            pl.BlockSpec((_RB, 16), lambda i: (i, 0)),
            pl.BlockSpec((_RB, 1), lambda i: (i, 0)),
            pl.BlockSpec((256, 128), lambda i: (0, 0)),
            pl.BlockSpec((1, 128), lambda i: (0, 0)),
            pl.BlockSpec((256, 128), lambda i: (0, 0)),
            pl.BlockSpec((1, 128), lambda i: (0, 0)),
            pl.BlockSpec((3, 6), lambda i: (0, 0)),
            pl.BlockSpec((1, 6), lambda i: (0, 0)),
        ],
        out_specs=[
            pl.BlockSpec((_RB, 128), lambda i: (i, 0)),
            pl.BlockSpec((_RB, 128), lambda i: (i, 0)),
            pl.BlockSpec((_RB, 6), lambda i: (i, 0)),
        ],
        out_shape=[
            jax.ShapeDtypeStruct((N, 128), jnp.float32),
            jax.ShapeDtypeStruct((N, 128), jnp.float32),
            jax.ShapeDtypeStruct((N, 6), jnp.float32),
        ])(acc1a[:, 0:N], acc1b[:, 0:N], acc2[:, 0:N], xnq, xnq, xnq, xnq,
           hs, dinv, Wmu, bmu, Wls, bls, Wn2, bn2)


def kernel(x, edge_index, W1, b1, W_mu, b_mu, W_ls, b_ls, W_n, b_n, W_n2, b_n2):
    src = edge_index[0].astype(jnp.int32)
    dst = edge_index[1].astype(jnp.int32)
    srcx2d = jnp.concatenate([src, src + N]).reshape(2 * E // CH, CH)
    dst2d = dst.reshape(E // CH, CH)

    degp = _deg_kernel(dst2d)
    v0s, dinv = _tc1(degp[0, 0:N, 0:1], x)
    acc0 = _gp64(v0s, srcx2d, dst2d)
    W1q = jnp.stack([W1[:, 64 * q:64 * (q + 1)] for q in range(4)])
    b1q = jnp.stack([b1[64 * q:64 * (q + 1)].reshape(1, -1) for q in range(4)])
    xnq, hs = _tc2(acc0, v0s, dinv, W1q, b1q, W_n, b_n.reshape(1, -1))
    acc1a = _gp64(xnq, srcx2d, dst2d)
    srcx2d_hi = srcx2d + 2 * N + (acc1a[0, 0, 0] * 0.0).astype(jnp.int32)
    acc1b = _gp64(xnq, srcx2d_hi, dst2d)
    acc2 = _gp16(hs, srcx2d, dst2d)
    return _tc3(acc1a, acc1b, acc2, xnq, hs, dinv, W_mu, b_mu.reshape(1, -1),
                W_ls, b_ls.reshape(1, -1), W_n2, b_n2.reshape(1, -1))


# speedup vs baseline: 26.4047x; 1.0340x over previous
import jax
import jax.numpy as jnp
from jax import lax
from jax.experimental import pallas as pl
from jax.experimental.pallas import tpu as pltpu
from jax.experimental.pallas import tpu_sc as plsc

N = 10000
E = 320000
N_PAD = 10240
CH = 100
ROWS_PER_TILE = (E // CH) // 16
RPS = N_PAD // 16
ZR = 128


def _make_prop(W, gather, trows):
    mesh = plsc.VectorSubcoreMesh(core_axis_name="c", subcore_axis_name="s")
    qn = W // 16
    RPT = ROWS_PER_TILE

    if gather:
        idx_scratch = [pltpu.VMEM((8, CH), jnp.int32) for _ in range(4)]
    else:
        idx_scratch = [pltpu.VMEM((RPT, CH), jnp.int32)]
    scratch_types = idx_scratch + [
        pltpu.VMEM((CH, W), jnp.float32) for _ in range(8)] + [
        pltpu.VMEM((ZR, W), jnp.float32),
        pltpu.VMEM_SHARED((N_PAD, W), jnp.float32),
        pltpu.SemaphoreType.DMA,
        pltpu.SemaphoreType.DMA,
        pltpu.SemaphoreType.DMA,
        pltpu.SemaphoreType.DMA,
        pltpu.SemaphoreType.DMA,
        pltpu.SemaphoreType.DMA,
    ]

    def body(*refs):
        if gather:
            (table, srcx, dst2d, out, sidxA, didxA, sidxB, didxB) = refs[:8]
            ring = refs[8:16]
            (zbuf, acc, gsa, gsb, ssa, ssb, semz, isem) = refs[16:]
        else:
            (dst2d, out, didx) = refs[:3]
            ring = refs[3:11]
            (zbuf, acc, gsa, gsb, ssa, ssb, semz, isem) = refs[11:]
        c = lax.axis_index("c")
        s = lax.axis_index("s")

        def zstore(i, _):
            zbuf[i // qn, pl.ds((i % qn) * 16, 16)] = jnp.zeros((16,), jnp.float32)
            return 0
        lax.fori_loop(0, ZR * qn, zstore, 0)
        for t in range(RPS // ZR):
            pltpu.async_copy(zbuf, acc.at[pl.ds(s * RPS + t * ZR, ZR)], semz)
        if not gather:
            for r in ring:
                def ostore(i, _, _r=r):
                    _r[i // qn, pl.ds((i % qn) * 16, 16)] = jnp.ones(
                        (16,), jnp.float32)
                    return 0
                lax.fori_loop(0, CH * qn, ostore, 0)

        drow0 = s * RPT
        if gather:
            srow0 = c * (E // CH) + s * RPT
            pltpu.sync_copy(srcx.at[pl.ds(srow0, 8)], sidxA)
            pltpu.sync_copy(dst2d.at[pl.ds(drow0, 8)], didxA)
            pltpu.async_copy(srcx.at[pl.ds(srow0 + 8, 8)], sidxB, isem)
            pltpu.async_copy(dst2d.at[pl.ds(drow0 + 8, 8)], didxB, isem)
        else:
            pltpu.sync_copy(dst2d.at[pl.ds(drow0, RPT)], didx)
        if gather:
            for q in range(4):
                pltpu.async_copy(table.at[sidxA.at[q]], ring[q], gsa)
        for t in range(RPS // ZR):
            pltpu.make_async_copy(zbuf, acc.at[pl.ds(s * RPS + t * ZR, ZR)],
                                  semz).wait()
        plsc.subcore_barrier()

        if gather:
            NBODY = RPT // 8

            def one_body(b, sidx, didx, sidx_o, didx_o):
                for q in range(4):
                    pltpu.make_async_copy(table.at[sidx.at[q]],
                                          ring[q], gsa).wait()
                for q in range(4):
                    pltpu.async_copy(ring[q], acc.at[didx.at[q]],
                                     ssa, add=True)
                @pl.when(b > 0)
                def _():
                    for q in range(4):
                        pltpu.make_async_copy(ring[4 + q],
                                              acc.at[didx_o.at[4 + q]],
                                              ssb).wait()
                    nrow = jnp.minimum(8 * (b + 1), RPT - 8)
                    pltpu.async_copy(srcx.at[pl.ds(srow0 + nrow, 8)],
                                     sidx_o, isem)
                    pltpu.async_copy(dst2d.at[pl.ds(drow0 + nrow, 8)],
                                     didx_o, isem)
                for q in range(4):
                    pltpu.async_copy(table.at[sidx.at[4 + q]],
                                     ring[4 + q], gsb)
                for q in range(4):
                    pltpu.make_async_copy(table.at[sidx.at[4 + q]],
                                          ring[4 + q], gsb).wait()
                for q in range(4):
                    pltpu.async_copy(ring[4 + q], acc.at[didx.at[4 + q]],
                                     ssb, add=True)
                for q in range(4):
                    pltpu.make_async_copy(ring[q],
                                          acc.at[didx.at[q]], ssa).wait()
                pltpu.make_async_copy(srcx.at[pl.ds(srow0, 8)], sidx_o,
                                      isem).wait()
                pltpu.make_async_copy(dst2d.at[pl.ds(drow0, 8)], didx_o,
                                      isem).wait()
                for q in range(4):
                    pltpu.async_copy(table.at[sidx_o.at[q]], ring[q], gsa)

            def bodyfn(b, _):
                @pl.when(b % 2 == 0)
                def _():
                    one_body(b, sidxA, didxA, sidxB, didxB)

                @pl.when(b % 2 == 1)
                def _():
                    one_body(b, sidxB, didxB, sidxA, didxA)
                return 0
            lax.fori_loop(0, NBODY, bodyfn, 0)
            for q in range(4):
                pltpu.make_async_copy(table.at[sidxA.at[q]],
                                      ring[q], gsa).wait()
            for q in range(4):
                pltpu.make_async_copy(ring[4 + q],
                                      acc.at[didxA.at[4 + q]],
                                      ssb).wait()
        else:
            def blk(b, _):
                for q in range(8):
                    pltpu.async_copy(ring[q], acc.at[didx.at[8 * b + q]],
                                     ssa, add=True)
                for q in range(8):
                    pltpu.make_async_copy(ring[q],
                                          acc.at[didx.at[8 * b + q]],
                                          ssa).wait()
                return 0
            lax.fori_loop(0, RPT // 8, blk, 0)

        plsc.subcore_barrier()
        for t in range(RPS // ZR):
            pltpu.async_copy(acc.at[pl.ds(s * RPS + t * ZR, ZR)],
                             out.at[c, pl.ds(s * RPS + t * ZR, ZR)], semz)
        for t in range(RPS // ZR):
            pltpu.make_async_copy(acc.at[pl.ds(s * RPS + t * ZR, ZR)],
                                  out.at[c, pl.ds(s * RPS + t * ZR, ZR)],
                                  semz).wait()

    return pl.kernel(
        body,
        out_type=jax.ShapeDtypeStruct((2, N_PAD, W), jnp.float32),
        mesh=mesh,
        scratch_types=scratch_types,
        compiler_params=pltpu.CompilerParams(use_tc_tiling_on_sc=False),
    )


_deg_kernel = _make_prop(16, gather=False, trows=None)
_gp16 = _make_prop(16, gather=True, trows=2 * N)
_gp64 = _make_prop(64, gather=True, trows=4 * N)

_RB = 2000
_NB = N // _RB


def _tc1(degcol, x):
    def body(deg_ref, x_ref, v0_ref, dinv_ref):
        dinv = lax.rsqrt(deg_ref[...] + 1.0)
        xd = x_ref[...] * dinv
        v0_ref[...] = jnp.concatenate(
            [xd[:, :64], xd[:, 64:], jnp.zeros((2 * N, 64), jnp.float32)],
            axis=0)
        dinv_ref[...] = dinv
    return pl.pallas_call(body, out_shape=[
        jax.ShapeDtypeStruct((4 * N, 64), jnp.float32),
        jax.ShapeDtypeStruct((N, 1), jnp.float32),
    ])(degcol, x)


def _tc2(acc0, v0s, dinv, W1, b1, Wn, bn):
    def body(a_ref, va_ref, vb_ref, d_ref, w1_ref, b1_ref, wn_ref, bn_ref,
             out_ref, h_ref):
        dinv = d_ref[...]
        s0 = (jnp.concatenate([a_ref[0], a_ref[1]], axis=1)
              + jnp.concatenate([va_ref[...], vb_ref[...]], axis=1))
        agg0 = dinv * s0
        xn = jnp.maximum(
            jnp.dot(agg0, w1_ref[0], preferred_element_type=jnp.float32)
            + b1_ref[0], 0.0)
        h = jnp.maximum(
            jnp.dot(agg0, wn_ref[...], preferred_element_type=jnp.float32)
            + bn_ref[...], 0.0)
        out_ref[...] = dinv * xn
        h_ref[...] = dinv * jnp.concatenate(
            [h, jnp.zeros((_RB, 13), jnp.float32)], axis=1)
    return pl.pallas_call(
        body,
        grid=(4, _NB),
        in_specs=[
            pl.BlockSpec((2, _RB, 64), lambda j, i: (0, i, 0)),
            pl.BlockSpec((_RB, 64), lambda j, i: (i, 0)),
            pl.BlockSpec((_RB, 64), lambda j, i: (_NB + i, 0)),
            pl.BlockSpec((_RB, 1), lambda j, i: (i, 0)),
            pl.BlockSpec((1, 128, 64), lambda j, i: (j, 0, 0)),
            pl.BlockSpec((1, 1, 64), lambda j, i: (j, 0, 0)),
            pl.BlockSpec((128, 3), lambda j, i: (0, 0)),
            pl.BlockSpec((1, 3), lambda j, i: (0, 0)),
        ],
        out_specs=[
            pl.BlockSpec((_RB, 64), lambda j, i: (j * _NB + i, 0)),
            pl.BlockSpec((_RB, 16), lambda j, i: ((j % 2) * _NB + i, 0)),
        ],
        out_shape=[
            jax.ShapeDtypeStruct((4 * N, 64), jnp.float32),
            jax.ShapeDtypeStruct((2 * N, 16), jnp.float32),
        ])(acc0, v0s, v0s, dinv, W1, b1, Wn, bn)


def _tc3(acc1a, acc1b, acc2, xnq, hs, dinv, Wmu, bmu, Wls, bls, Wn2, bn2):
    def body(a1_ref, b1_ref, a2_ref, x0_ref, x1_ref, x2_ref, x3_ref, h_ref,
             d_ref, wmu_ref, bmu_ref, wls_ref, bls_ref, wn2_ref, bn2_ref,
             mu_ref, ls_ref, node_ref):
        dinv = d_ref[...]
        ga = dinv * (jnp.concatenate([a1_ref[0], a1_ref[1]], axis=1)
                     + jnp.concatenate([x0_ref[...], x1_ref[...]], axis=1))
        gb = dinv * (jnp.concatenate([b1_ref[0], b1_ref[1]], axis=1)
                     + jnp.concatenate([x2_ref[...], x3_ref[...]], axis=1))
        g2 = (dinv * (a2_ref[0] + h_ref[...]))[:, 0:3]
        mu_ref[...] = (
            jnp.dot(ga, wmu_ref[:128], preferred_element_type=jnp.float32)
            + jnp.dot(gb, wmu_ref[128:], preferred_element_type=jnp.float32)
            + bmu_ref[...])
        ls_ref[...] = (
            jnp.dot(ga, wls_ref[:128], preferred_element_type=jnp.float32)
            + jnp.dot(gb, wls_ref[128:], preferred_element_type=jnp.float32)
            + bls_ref[...])
        node_ref[...] = (
            jnp.dot(g2, wn2_ref[...], preferred_element_type=jnp.float32)
            + bn2_ref[...])
    return pl.pallas_call(
        body,
        grid=(_NB,),
        in_specs=[
            pl.BlockSpec((2, _RB, 64), lambda i: (0, i, 0)),
            pl.BlockSpec((2, _RB, 64), lambda i: (0, i, 0)),
            pl.BlockSpec((2, _RB, 16), lambda i: (0, i, 0)),
            pl.BlockSpec((_RB, 64), lambda i: (i, 0)),
            pl.BlockSpec((_RB, 64), lambda i: (_NB + i, 0)),
            pl.BlockSpec((_RB, 64), lambda i: (2 * _NB + i, 0)),
            pl.BlockSpec((_RB, 64), lambda i: (3 * _NB + i, 0)),
            pl.BlockSpec((_RB, 16), lambda i: (i, 0)),
            pl.BlockSpec((_RB, 1), lambda i: (i, 0)),
            pl.BlockSpec((256, 128), lambda i: (0, 0)),
            pl.BlockSpec((1, 128), lambda i: (0, 0)),
            pl.BlockSpec((256, 128), lambda i: (0, 0)),
            pl.BlockSpec((1, 128), lambda i: (0, 0)),
            pl.BlockSpec((3, 6), lambda i: (0, 0)),
            pl.BlockSpec((1, 6), lambda i: (0, 0)),
        ],
        out_specs=[
            pl.BlockSpec((_RB, 128), lambda i: (i, 0)),
            pl.BlockSpec((_RB, 128), lambda i: (i, 0)),
            pl.BlockSpec((_RB, 6), lambda i: (i, 0)),
        ],
        out_shape=[
            jax.ShapeDtypeStruct((N, 128), jnp.float32),
            jax.ShapeDtypeStruct((N, 128), jnp.float32),
            jax.ShapeDtypeStruct((N, 6), jnp.float32),
        ])(acc1a, acc1b, acc2, xnq, xnq, xnq, xnq,
           hs, dinv, Wmu, bmu, Wls, bls, Wn2, bn2)


def kernel(x, edge_index, W1, b1, W_mu, b_mu, W_ls, b_ls, W_n, b_n, W_n2, b_n2):
    src = edge_index[0].astype(jnp.int32)
    dst = edge_index[1].astype(jnp.int32)
    srcx2d = jnp.concatenate([src, src + N]).reshape(2 * E // CH, CH)
    dst2d = dst.reshape(E // CH, CH)

    degp = _deg_kernel(dst2d)
    v0s, dinv = _tc1(degp[0, 0:N, 0:1], x)
    acc0 = _gp64(v0s, srcx2d, dst2d)
    W1q = jnp.stack([W1[:, 64 * q:64 * (q + 1)] for q in range(4)])
    b1q = jnp.stack([b1[64 * q:64 * (q + 1)].reshape(1, -1) for q in range(4)])
    xnq, hs = _tc2(acc0, v0s, dinv, W1q, b1q, W_n, b_n.reshape(1, -1))
    acc2 = _gp16(hs, srcx2d, dst2d)
    acc1a = _gp64(xnq, srcx2d, dst2d)
    srcx2d_hi = srcx2d + 2 * N + (acc1a[0, 0, 0] * 0.0).astype(jnp.int32)
    acc1b = _gp64(xnq, srcx2d_hi, dst2d)
    return _tc3(acc1a, acc1b, acc2, xnq, hs, dinv, W_mu, b_mu.reshape(1, -1),
                W_ls, b_ls.reshape(1, -1), W_n2, b_n2.reshape(1, -1))
